# baseline (speedup 1.0000x reference)
import functools

import jax
import jax.numpy as jnp
from jax import lax
from jax.experimental import pallas as pl
from jax.experimental.pallas import tpu as pltpu
from jax.experimental.pallas import tpu_sc as plsc

NC = 2
NS = 16
NW = NC * NS
L = 16
G = 64
NH = 2

_HIGH = jax.lax.Precision.HIGHEST


def _sc_mesh():
  return plsc.VectorSubcoreMesh(
      core_axis_name="c", subcore_axis_name="s",
      num_cores=NC, num_subcores=NS)


@functools.lru_cache(maxsize=None)
def _make_deg(NP, CH, K):
  RPT = NP // NS
  RB = 128
  assert RPT % RB == 0

  @functools.partial(
      pl.kernel,
      out_type=jax.ShapeDtypeStruct((NC, NP, L), jnp.float32),
      mesh=_sc_mesh(),
      compiler_params=pltpu.CompilerParams(use_tc_tiling_on_sc=False),
      scratch_types=[
          pltpu.VMEM((CH, K), jnp.int32),
          pltpu.VMEM((K, L), jnp.float32),
          pltpu.VMEM((RB, L), jnp.float32),
          pltpu.VMEM_SHARED((NP, L), jnp.float32),
      ],
  )
  def deg_k(dstm, out, idx_v, ones_v, stage_v, acc):
    c = lax.axis_index("c")
    s = lax.axis_index("s")
    ones16 = jnp.ones((L,), jnp.float32)
    z16 = jnp.zeros((L,), jnp.float32)

    def fill_ones(r, carry):
      ones_v[r, :] = ones16
      return carry
    lax.fori_loop(0, K, fill_ones, 0)

    def fill_zero(r, carry):
      stage_v[r, :] = z16
      return carry
    lax.fori_loop(0, RB, fill_zero, 0)

    row0 = s * RPT
    def zero_chunk(t, carry):
      pltpu.sync_copy(stage_v, acc.at[pl.ds(row0 + t * RB, RB)])
      return carry
    lax.fori_loop(0, RPT // RB, zero_chunk, 0)
    plsc.subcore_barrier()

    pltpu.sync_copy(dstm.at[c, s], idx_v)
    def chunk(j, carry):
      pltpu.sync_copy(ones_v, acc.at[idx_v.at[j]], add=True)
      return carry
    lax.fori_loop(0, CH, chunk, 0)
    plsc.subcore_barrier()

    def readout(t, carry):
      r = row0 + t * RB
      pltpu.sync_copy(acc.at[pl.ds(r, RB)], stage_v)
      pltpu.sync_copy(stage_v, out.at[c, pl.ds(r, RB)])
      return carry
    lax.fori_loop(0, RPT // RB, readout, 0)

  return deg_k


@functools.lru_cache(maxsize=None)
def _make_prop(N, NP, HH, CH, K):
  RPT = NP // NS
  RB = 128
  NBUF = 4
  assert RPT % RB == 0
  assert HH % L == 0 and (HH * 4) % 64 == 0
  assert CH % NBUF == 0

  @functools.partial(
      pl.kernel,
      out_type=[jax.ShapeDtypeStruct((NC, NP, HH), jnp.float32)] * NH,
      mesh=_sc_mesh(),
      compiler_params=pltpu.CompilerParams(use_tc_tiling_on_sc=False),
      scratch_types=[
          pltpu.VMEM((CH, K), jnp.int32),
          pltpu.VMEM((CH, K), jnp.int32),
          [pltpu.VMEM((K, HH), jnp.float32)] * 4,
          pltpu.VMEM((RB, HH), jnp.float32),
          pltpu.VMEM((RB, HH), jnp.float32),
          pltpu.VMEM_SHARED((NP, HH), jnp.float32),
          [pltpu.SemaphoreType.DMA] * 4,
          [pltpu.SemaphoreType.DMA] * 4,
      ],
  )
  def prop_k(h0, h1, srcm, dstm, out0, out1, idx_s, idx_d, rows, zero_v,
             stage_v, acc, gsems, ssems):
    tables = (h0, h1)
    outs = (out0, out1)
    c = lax.axis_index("c")
    s = lax.axis_index("s")
    z16 = jnp.zeros((L,), jnp.float32)

    def fill_zero(r, carry):
      for q in range(HH // L):
        zero_v[r, pl.ds(q * L, L)] = z16
      return carry
    lax.fori_loop(0, RB, fill_zero, 0)

    pltpu.sync_copy(srcm.at[c, s], idx_s)
    pltpu.sync_copy(dstm.at[c, s], idx_d)

    row0 = s * RPT
    for hh in range(NH):
      def zero_chunk(t, carry):
        pltpu.sync_copy(zero_v, acc.at[pl.ds(row0 + t * RB, RB)])
        return carry
      lax.fori_loop(0, RPT // RB, zero_chunk, 0)
      plsc.subcore_barrier()

      table = tables[hh]
      pltpu.async_copy(table.at[idx_s.at[0]], rows[0], gsems[0])
      pltpu.async_copy(table.at[idx_s.at[1]], rows[1], gsems[1])

      def quad(t, carry):
        for b in range(4):
          j = 4 * t + b
          bb = (b + 2) % 4
          pltpu.make_async_copy(table.at[idx_s.at[j]], rows[b],
                                gsems[b]).wait()

          @pl.when(j - 2 >= 0)
          def _():
            pltpu.make_async_copy(rows[bb], acc.at[idx_d.at[j - 2]],
                                  ssems[bb]).wait()
          pltpu.async_copy(rows[b], acc.at[idx_d.at[j]], ssems[b], add=True)

          @pl.when(j + 2 < CH)
          def _():
            pltpu.async_copy(table.at[idx_s.at[j + 2]], rows[bb], gsems[bb])
        return carry
      lax.fori_loop(0, CH // 4, quad, 0)
      for b in (2, 3):
        pltpu.make_async_copy(rows[b], acc.at[idx_d.at[CH - 4 + b]],
                              ssems[b]).wait()
      plsc.subcore_barrier()

      def readout(t, carry):
        r = row0 + t * RB
        pltpu.sync_copy(acc.at[pl.ds(r, RB)], stage_v)
        pltpu.sync_copy(stage_v, outs[hh].at[c, pl.ds(r, RB)])
        return carry
      lax.fori_loop(0, RPT // RB, readout, 0)

  return prop_k


def _tc1_body(x_ref, w0_ref, w1_ref, degc_ref, o0_ref, o1_ref, dis_ref):
  n = x_ref.shape[0]
  deg = 1.0 + degc_ref[0, :n] + degc_ref[1, :n]
  dis = lax.rsqrt(jnp.maximum(deg, 1.0))
  dis_ref[:] = dis
  for w_ref, o_ref in ((w0_ref, o0_ref), (w1_ref, o1_ref)):
    h = jnp.dot(x_ref[:, :], w_ref[:, :],
                preferred_element_type=jnp.float32)
    o_ref[:, :] = h * dis[:, None]


def _onehot_blk(batch_blk):
  n = batch_blk.shape[0]
  ids = lax.broadcasted_iota(jnp.int32, (n, G), 1)
  return (batch_blk[:, None] == ids).astype(jnp.float32)


def _gn_stats(onehot, x, st_ref):
  sx = jnp.concatenate([x, x * x], axis=1)
  st_ref[:, :] += lax.dot_general(onehot, sx, (((0,), (0,)), ((), ())),
                                  preferred_element_type=jnp.float32,
                                  precision=_HIGH)


def _gn_apply(onehot, x, st, inv_cnt, gw, gb, ga):
  hh = x.shape[1]
  seg = st * inv_cnt[:, None]
  mean = seg[:, :hh]
  ex2 = seg[:, hh:]
  var = ex2 - (2.0 * ga - ga * ga) * mean * mean
  istd = lax.rsqrt(var + 1e-5)
  bc = jnp.dot(onehot, jnp.concatenate([mean * ga, istd], axis=1),
               preferred_element_type=jnp.float32, precision=_HIGH)
  xm = x - bc[:, :hh]
  return jax.nn.relu(gw * xm * bc[:, hh:] + gb)


def _block_x(s_ref, ht_ref, dis, b_ref, q):
  hh = ht_ref.shape[1]
  return ((s_ref[0] + s_ref[1] + ht_ref[:, :]) * dis[:, None]
          + b_ref[pl.ds(q * hh, hh)])


def _conv_tail_body(s0_ref, s1_ref, ht0_ref, ht1_ref, dis_ref, batch_ref,
                    b_ref, gw_ref, gb_ref, ga_ref, w2_ref, o0_ref, o1_ref,
                    st0_ref, st1_ref, cnt_ref):
  ph = pl.program_id(0)
  i = pl.program_id(1)
  n, hh = ht0_ref.shape
  dis = dis_ref[0, 0, :]
  onehot = _onehot_blk(batch_ref[0, 0, :])
  xs = [_block_x(s0_ref, ht0_ref, dis, b_ref, 0),
        _block_x(s1_ref, ht1_ref, dis, b_ref, 1)]

  @pl.when(ph == 0)
  def _():
    @pl.when(i == 0)
    def _():
      st0_ref[:, :] = jnp.zeros_like(st0_ref)
      st1_ref[:, :] = jnp.zeros_like(st1_ref)
      cnt_ref[:] = jnp.zeros_like(cnt_ref)
    cnt_ref[:] += jnp.sum(onehot, axis=0)
    _gn_stats(onehot, xs[0], st0_ref)
    _gn_stats(onehot, xs[1], st1_ref)

  @pl.when(ph == 1)
  def _():
    inv_cnt = 1.0 / jnp.maximum(cnt_ref[:], 1.0)
    gs = [_gn_apply(onehot, xs[q], st_ref[:, :], inv_cnt,
                    gw_ref[pl.ds(q * hh, hh)], gb_ref[pl.ds(q * hh, hh)],
                    ga_ref[pl.ds(q * hh, hh)])
          for q, st_ref in ((0, st0_ref), (1, st1_ref))]
    for q, o_ref in ((0, o0_ref), (1, o1_ref)):
      acc = jnp.dot(gs[0], w2_ref[:hh, q * hh:(q + 1) * hh],
                    preferred_element_type=jnp.float32)
      acc = acc + jnp.dot(gs[1], w2_ref[hh:, q * hh:(q + 1) * hh],
                          preferred_element_type=jnp.float32)
      o_ref[:, :] = acc * dis[:, None]


def _final_body(s0_ref, s1_ref, ht0_ref, ht1_ref, dis_ref, batch_ref,
                b_ref, gw_ref, gb_ref, ga_ref, f1w_ref, f1b_ref, f2w_ref,
                f2b_ref, o_ref, st0_ref, st1_ref, cnt_ref, pool_ref):
  ph = pl.program_id(0)
  i = pl.program_id(1)
  nb = pl.num_programs(1)
  n, hh = ht0_ref.shape
  dis = dis_ref[0, 0, :]
  onehot = _onehot_blk(batch_ref[0, 0, :])
  xs = [_block_x(s0_ref, ht0_ref, dis, b_ref, 0),
        _block_x(s1_ref, ht1_ref, dis, b_ref, 1)]

  @pl.when(ph == 0)
  def _():
    @pl.when(i == 0)
    def _():
      st0_ref[:, :] = jnp.zeros_like(st0_ref)
      st1_ref[:, :] = jnp.zeros_like(st1_ref)
      cnt_ref[:] = jnp.zeros_like(cnt_ref)
      pool_ref[:, :] = jnp.zeros_like(pool_ref)
    cnt_ref[:] += jnp.sum(onehot, axis=0)
    _gn_stats(onehot, xs[0], st0_ref)
    _gn_stats(onehot, xs[1], st1_ref)

  @pl.when(ph == 1)
  def _():
    inv_cnt = 1.0 / jnp.maximum(cnt_ref[:], 1.0)
    for q, st_ref in ((0, st0_ref), (1, st1_ref)):
      gq = _gn_apply(onehot, xs[q], st_ref[:, :], inv_cnt,
                     gw_ref[pl.ds(q * hh, hh)], gb_ref[pl.ds(q * hh, hh)],
                     ga_ref[pl.ds(q * hh, hh)])
      pool_ref[:, pl.ds(q * hh, hh)] += lax.dot_general(
          onehot, gq, (((0,), (0,)), ((), ())),
          preferred_element_type=jnp.float32, precision=_HIGH)

    @pl.when(i == nb - 1)
    def _():
      g = pool_ref[:, :] * inv_cnt[:, None]
      g = jax.nn.relu(jnp.dot(g, f1w_ref[:, :],
                              preferred_element_type=jnp.float32)
                      + f1b_ref[:])
      o_ref[:, :] = jnp.dot(g, f2w_ref[:, :],
                            preferred_element_type=jnp.float32) + f2b_ref[:]


def kernel(x, edge_index, batch, W1, b1, gn1_w, gn1_b, gn1_a,
           W2, b2, gn2_w, gn2_b, gn2_a, fc1_W, fc1_b, fc2_W, fc2_b):
  N, _ = x.shape
  H = W1.shape[1]
  HH = H // NH
  E = edge_index.shape[1]
  per = E // NW
  assert E % NW == 0 and H % NH == 0
  K = 50
  assert per % K == 0
  CH = per // K
  NP = NS * 128 * ((N + NS * 128 - 1) // (NS * 128))

  srcm = edge_index[0].reshape(NC, NS, CH, K)
  dstm = edge_index[1].reshape(NC, NS, CH, K)

  degp = _make_deg(NP, CH, K)(dstm)
  degc = degp[:, :, 0]

  prop = _make_prop(N, NP, HH, CH, K)

  ht1a, ht1b, dis = pl.pallas_call(
      _tc1_body,
      out_shape=[jax.ShapeDtypeStruct((N, HH), jnp.float32)] * NH
      + [jax.ShapeDtypeStruct((N,), jnp.float32)],
  )(x, W1[:, :HH], W1[:, HH:], degc)

  s1a, s1b = prop(ht1a, ht1b, srcm, dstm)

  BR = 1000
  NB = N // BR
  assert N % BR == 0
  dis3 = dis.reshape(NB, 1, BR)
  batch3 = batch.reshape(NB, 1, BR)
  s_spec = pl.BlockSpec((NC, BR, HH), lambda p, i: (0, i, 0))
  h_spec = pl.BlockSpec((BR, HH), lambda p, i: (i, 0))
  v_spec = pl.BlockSpec((1, 1, BR), lambda p, i: (i, 0, 0))
  p_spec = pl.BlockSpec((H,), lambda p, i: (0,))
  w_spec = pl.BlockSpec((H, H), lambda p, i: (0, 0))
  stats_t = [pltpu.VMEM((G, 2 * HH), jnp.float32),
             pltpu.VMEM((G, 2 * HH), jnp.float32),
             pltpu.VMEM((G,), jnp.float32)]

  ht2a, ht2b = pl.pallas_call(
      _conv_tail_body,
      grid=(2, NB),
      in_specs=[s_spec, s_spec, h_spec, h_spec, v_spec, v_spec,
                p_spec, p_spec, p_spec, p_spec, w_spec],
      out_specs=[h_spec, h_spec],
      out_shape=[jax.ShapeDtypeStruct((N, HH), jnp.float32)] * NH,
      scratch_shapes=stats_t,
  )(s1a, s1b, ht1a, ht1b, dis3, batch3, b1, gn1_w, gn1_b, gn1_a, W2)

  s2a, s2b = prop(ht2a, ht2b, srcm, dstm)

  C = fc2_W.shape[1]
  out = pl.pallas_call(
      _final_body,
      grid=(2, NB),
      in_specs=[s_spec, s_spec, h_spec, h_spec, v_spec, v_spec,
                p_spec, p_spec, p_spec, p_spec,
                pl.BlockSpec(fc1_W.shape, lambda p, i: (0, 0)),
                pl.BlockSpec(fc1_b.shape, lambda p, i: (0,)),
                pl.BlockSpec(fc2_W.shape, lambda p, i: (0, 0)),
                pl.BlockSpec(fc2_b.shape, lambda p, i: (0,))],
      out_specs=pl.BlockSpec((G, C), lambda p, i: (0, 0)),
      out_shape=jax.ShapeDtypeStruct((G, C), jnp.float32),
      scratch_shapes=stats_t + [pltpu.VMEM((G, H), jnp.float32)],
  )(s2a, s2b, ht2a, ht2b, dis3, batch3, b2, gn2_w, gn2_b, gn2_a,
    fc1_W, fc1_b, fc2_W, fc2_b)
  return out


# speedup vs baseline: 1.1758x; 1.1758x over previous
import functools

import jax
import jax.numpy as jnp
from jax import lax
from jax.experimental import pallas as pl
from jax.experimental.pallas import tpu as pltpu
from jax.experimental.pallas import tpu_sc as plsc

NC = 2
NS = 16
NW = NC * NS
L = 16
G = 64
NH = 2

_HIGH = jax.lax.Precision.HIGHEST


def _sc_mesh():
  return plsc.VectorSubcoreMesh(
      core_axis_name="c", subcore_axis_name="s",
      num_cores=NC, num_subcores=NS)


@functools.lru_cache(maxsize=None)
def _make_deg(NP, CH, K):
  RPT = NP // NS
  RB = 128
  assert RPT % RB == 0

  @functools.partial(
      pl.kernel,
      out_type=jax.ShapeDtypeStruct((NC, NP, L), jnp.float32),
      mesh=_sc_mesh(),
      compiler_params=pltpu.CompilerParams(use_tc_tiling_on_sc=False),
      scratch_types=[
          pltpu.VMEM((CH, K), jnp.int32),
          pltpu.VMEM((K, L), jnp.float32),
          pltpu.VMEM((RB, L), jnp.float32),
          pltpu.VMEM_SHARED((NP, L), jnp.float32),
      ],
  )
  def deg_k(dstm, out, idx_v, ones_v, stage_v, acc):
    c = lax.axis_index("c")
    s = lax.axis_index("s")
    ones16 = jnp.ones((L,), jnp.float32)
    z16 = jnp.zeros((L,), jnp.float32)

    def fill_ones(r, carry):
      ones_v[r, :] = ones16
      return carry
    lax.fori_loop(0, K, fill_ones, 0)

    def fill_zero(r, carry):
      stage_v[r, :] = z16
      return carry
    lax.fori_loop(0, RB, fill_zero, 0)

    row0 = s * RPT
    def zero_chunk(t, carry):
      pltpu.sync_copy(stage_v, acc.at[pl.ds(row0 + t * RB, RB)])
      return carry
    lax.fori_loop(0, RPT // RB, zero_chunk, 0)
    plsc.subcore_barrier()

    pltpu.sync_copy(dstm.at[c, s], idx_v)
    def chunk(j, carry):
      pltpu.sync_copy(ones_v, acc.at[idx_v.at[j]], add=True)
      return carry
    lax.fori_loop(0, CH, chunk, 0)
    plsc.subcore_barrier()

    def readout(t, carry):
      r = row0 + t * RB
      pltpu.sync_copy(acc.at[pl.ds(r, RB)], stage_v)
      pltpu.sync_copy(stage_v, out.at[c, pl.ds(r, RB)])
      return carry
    lax.fori_loop(0, RPT // RB, readout, 0)

  return deg_k


@functools.lru_cache(maxsize=None)
def _make_prop(N, NP, HH, CH, K):
  RPT = NP // NS
  RB = 128
  assert RPT % RB == 0
  assert HH % L == 0 and (HH * 4) % 64 == 0
  assert CH % 2 == 0

  @functools.partial(
      pl.kernel,
      out_type=[jax.ShapeDtypeStruct((NC, NP, HH), jnp.float32)] * NH,
      mesh=_sc_mesh(),
      compiler_params=pltpu.CompilerParams(use_tc_tiling_on_sc=False),
      scratch_types=[
          pltpu.VMEM((CH, K), jnp.int32),
          pltpu.VMEM((CH, K), jnp.int32),
          pltpu.VMEM((K, HH), jnp.float32),
          pltpu.VMEM((K, HH), jnp.float32),
          pltpu.VMEM((RB, HH), jnp.float32),
          pltpu.VMEM((RB, HH), jnp.float32),
          pltpu.VMEM_SHARED((NP, HH), jnp.float32),
          pltpu.SemaphoreType.DMA,
          pltpu.SemaphoreType.DMA,
      ],
  )
  def prop_k(h0, h1, srcm, dstm, out0, out1, idx_s, idx_d, rows0, rows1,
             zero_v, stage_v, acc, sem0, sem1):
    tables = (h0, h1)
    outs = (out0, out1)
    c = lax.axis_index("c")
    s = lax.axis_index("s")
    z16 = jnp.zeros((L,), jnp.float32)

    def fill_zero(r, carry):
      for q in range(HH // L):
        zero_v[r, pl.ds(q * L, L)] = z16
      return carry
    lax.fori_loop(0, RB, fill_zero, 0)

    pltpu.sync_copy(srcm.at[c, s], idx_s)
    pltpu.sync_copy(dstm.at[c, s], idx_d)

    row0 = s * RPT
    for hh in range(NH):
      def zero_chunk(t, carry):
        pltpu.sync_copy(zero_v, acc.at[pl.ds(row0 + t * RB, RB)])
        return carry
      lax.fori_loop(0, RPT // RB, zero_chunk, 0)
      plsc.subcore_barrier()

      table = tables[hh]
      pltpu.async_copy(table.at[idx_s.at[0]], rows0, sem0)

      def pair(t, carry):
        j0 = 2 * t
        pltpu.async_copy(table.at[idx_s.at[j0 + 1]], rows1, sem1)
        pltpu.make_async_copy(table.at[idx_s.at[j0]], rows0, sem0).wait()
        pltpu.sync_copy(rows0, acc.at[idx_d.at[j0]], add=True)

        @pl.when(j0 + 2 < CH)
        def _():
          pltpu.async_copy(table.at[idx_s.at[j0 + 2]], rows0, sem0)
        pltpu.make_async_copy(table.at[idx_s.at[j0 + 1]], rows1, sem1).wait()
        pltpu.sync_copy(rows1, acc.at[idx_d.at[j0 + 1]], add=True)
        return carry
      lax.fori_loop(0, CH // 2, pair, 0)
      plsc.subcore_barrier()

      def readout(t, carry):
        r = row0 + t * RB
        pltpu.sync_copy(acc.at[pl.ds(r, RB)], stage_v)
        pltpu.sync_copy(stage_v, outs[hh].at[c, pl.ds(r, RB)])
        return carry
      lax.fori_loop(0, RPT // RB, readout, 0)

  return prop_k


def _tc1_body(x_ref, w0_ref, w1_ref, degc_ref, o0_ref, o1_ref, dis_ref):
  n = x_ref.shape[0]
  deg = 1.0 + degc_ref[0, :n] + degc_ref[1, :n]
  dis = lax.rsqrt(jnp.maximum(deg, 1.0))
  dis_ref[:] = dis
  for w_ref, o_ref in ((w0_ref, o0_ref), (w1_ref, o1_ref)):
    h = jnp.dot(x_ref[:, :], w_ref[:, :],
                preferred_element_type=jnp.float32)
    o_ref[:, :] = h * dis[:, None]


def _onehot_blk(batch_blk):
  n = batch_blk.shape[0]
  ids = lax.broadcasted_iota(jnp.int32, (n, G), 1)
  return (batch_blk[:, None] == ids).astype(jnp.float32)


def _gn_stats(onehot, x, st_ref):
  sx = jnp.concatenate([x, x * x], axis=1)
  st_ref[:, :] += lax.dot_general(onehot, sx, (((0,), (0,)), ((), ())),
                                  preferred_element_type=jnp.float32,
                                  precision=_HIGH)


def _gn_apply(onehot, x, st, inv_cnt, gw, gb, ga):
  hh = x.shape[1]
  seg = st * inv_cnt[:, None]
  mean = seg[:, :hh]
  ex2 = seg[:, hh:]
  var = ex2 - (2.0 * ga - ga * ga) * mean * mean
  istd = lax.rsqrt(var + 1e-5)
  bc = jnp.dot(onehot, jnp.concatenate([mean * ga, istd], axis=1),
               preferred_element_type=jnp.float32, precision=_HIGH)
  xm = x - bc[:, :hh]
  return jax.nn.relu(gw * xm * bc[:, hh:] + gb)


def _block_x(s_ref, ht_ref, dis, b_ref, q):
  hh = ht_ref.shape[1]
  return ((s_ref[0] + s_ref[1] + ht_ref[:, :]) * dis[:, None]
          + b_ref[pl.ds(q * hh, hh)])


def _conv_tail_body(s0_ref, s1_ref, ht0_ref, ht1_ref, dis_ref, batch_ref,
                    b_ref, gw_ref, gb_ref, ga_ref, w2_ref, o0_ref, o1_ref,
                    st0_ref, st1_ref, cnt_ref):
  ph = pl.program_id(0)
  i = pl.program_id(1)
  n, hh = ht0_ref.shape
  dis = dis_ref[0, 0, :]
  onehot = _onehot_blk(batch_ref[0, 0, :])
  xs = [_block_x(s0_ref, ht0_ref, dis, b_ref, 0),
        _block_x(s1_ref, ht1_ref, dis, b_ref, 1)]

  @pl.when(ph == 0)
  def _():
    @pl.when(i == 0)
    def _():
      st0_ref[:, :] = jnp.zeros_like(st0_ref)
      st1_ref[:, :] = jnp.zeros_like(st1_ref)
      cnt_ref[:] = jnp.zeros_like(cnt_ref)
    cnt_ref[:] += jnp.sum(onehot, axis=0)
    _gn_stats(onehot, xs[0], st0_ref)
    _gn_stats(onehot, xs[1], st1_ref)

  @pl.when(ph == 1)
  def _():
    inv_cnt = 1.0 / jnp.maximum(cnt_ref[:], 1.0)
    gs = [_gn_apply(onehot, xs[q], st_ref[:, :], inv_cnt,
                    gw_ref[pl.ds(q * hh, hh)], gb_ref[pl.ds(q * hh, hh)],
                    ga_ref[pl.ds(q * hh, hh)])
          for q, st_ref in ((0, st0_ref), (1, st1_ref))]
    for q, o_ref in ((0, o0_ref), (1, o1_ref)):
      acc = jnp.dot(gs[0], w2_ref[:hh, q * hh:(q + 1) * hh],
                    preferred_element_type=jnp.float32)
      acc = acc + jnp.dot(gs[1], w2_ref[hh:, q * hh:(q + 1) * hh],
                          preferred_element_type=jnp.float32)
      o_ref[:, :] = acc * dis[:, None]


def _final_body(s0_ref, s1_ref, ht0_ref, ht1_ref, dis_ref, batch_ref,
                b_ref, gw_ref, gb_ref, ga_ref, f1w_ref, f1b_ref, f2w_ref,
                f2b_ref, o_ref, st0_ref, st1_ref, cnt_ref, pool_ref):
  ph = pl.program_id(0)
  i = pl.program_id(1)
  nb = pl.num_programs(1)
  n, hh = ht0_ref.shape
  dis = dis_ref[0, 0, :]
  onehot = _onehot_blk(batch_ref[0, 0, :])
  xs = [_block_x(s0_ref, ht0_ref, dis, b_ref, 0),
        _block_x(s1_ref, ht1_ref, dis, b_ref, 1)]

  @pl.when(ph == 0)
  def _():
    @pl.when(i == 0)
    def _():
      st0_ref[:, :] = jnp.zeros_like(st0_ref)
      st1_ref[:, :] = jnp.zeros_like(st1_ref)
      cnt_ref[:] = jnp.zeros_like(cnt_ref)
      pool_ref[:, :] = jnp.zeros_like(pool_ref)
    cnt_ref[:] += jnp.sum(onehot, axis=0)
    _gn_stats(onehot, xs[0], st0_ref)
    _gn_stats(onehot, xs[1], st1_ref)

  @pl.when(ph == 1)
  def _():
    inv_cnt = 1.0 / jnp.maximum(cnt_ref[:], 1.0)
    for q, st_ref in ((0, st0_ref), (1, st1_ref)):
      gq = _gn_apply(onehot, xs[q], st_ref[:, :], inv_cnt,
                     gw_ref[pl.ds(q * hh, hh)], gb_ref[pl.ds(q * hh, hh)],
                     ga_ref[pl.ds(q * hh, hh)])
      pool_ref[:, pl.ds(q * hh, hh)] += lax.dot_general(
          onehot, gq, (((0,), (0,)), ((), ())),
          preferred_element_type=jnp.float32, precision=_HIGH)

    @pl.when(i == nb - 1)
    def _():
      g = pool_ref[:, :] * inv_cnt[:, None]
      g = jax.nn.relu(jnp.dot(g, f1w_ref[:, :],
                              preferred_element_type=jnp.float32)
                      + f1b_ref[:])
      o_ref[:, :] = jnp.dot(g, f2w_ref[:, :],
                            preferred_element_type=jnp.float32) + f2b_ref[:]


def kernel(x, edge_index, batch, W1, b1, gn1_w, gn1_b, gn1_a,
           W2, b2, gn2_w, gn2_b, gn2_a, fc1_W, fc1_b, fc2_W, fc2_b):
  N, _ = x.shape
  H = W1.shape[1]
  HH = H // NH
  E = edge_index.shape[1]
  per = E // NW
  assert E % NW == 0 and H % NH == 0
  K = 125
  assert per % K == 0
  CH = per // K
  NP = NS * 128 * ((N + NS * 128 - 1) // (NS * 128))

  srcm = edge_index[0].reshape(NC, NS, CH, K)
  dstm = edge_index[1].reshape(NC, NS, CH, K)

  degp = _make_deg(NP, CH, K)(dstm)
  degc = degp[:, :, 0]

  prop = _make_prop(N, NP, HH, CH, K)

  ht1a, ht1b, dis = pl.pallas_call(
      _tc1_body,
      out_shape=[jax.ShapeDtypeStruct((N, HH), jnp.float32)] * NH
      + [jax.ShapeDtypeStruct((N,), jnp.float32)],
  )(x, W1[:, :HH], W1[:, HH:], degc)

  s1a, s1b = prop(ht1a, ht1b, srcm, dstm)

  BR = 1000
  NB = N // BR
  assert N % BR == 0
  dis3 = dis.reshape(NB, 1, BR)
  batch3 = batch.reshape(NB, 1, BR)
  s_spec = pl.BlockSpec((NC, BR, HH), lambda p, i: (0, i, 0))
  h_spec = pl.BlockSpec((BR, HH), lambda p, i: (i, 0))
  v_spec = pl.BlockSpec((1, 1, BR), lambda p, i: (i, 0, 0))
  p_spec = pl.BlockSpec((H,), lambda p, i: (0,))
  w_spec = pl.BlockSpec((H, H), lambda p, i: (0, 0))
  stats_t = [pltpu.VMEM((G, 2 * HH), jnp.float32),
             pltpu.VMEM((G, 2 * HH), jnp.float32),
             pltpu.VMEM((G,), jnp.float32)]

  ht2a, ht2b = pl.pallas_call(
      _conv_tail_body,
      grid=(2, NB),
      in_specs=[s_spec, s_spec, h_spec, h_spec, v_spec, v_spec,
                p_spec, p_spec, p_spec, p_spec, w_spec],
      out_specs=[h_spec, h_spec],
      out_shape=[jax.ShapeDtypeStruct((N, HH), jnp.float32)] * NH,
      scratch_shapes=stats_t,
  )(s1a, s1b, ht1a, ht1b, dis3, batch3, b1, gn1_w, gn1_b, gn1_a, W2)

  s2a, s2b = prop(ht2a, ht2b, srcm, dstm)

  C = fc2_W.shape[1]
  out = pl.pallas_call(
      _final_body,
      grid=(2, NB),
      in_specs=[s_spec, s_spec, h_spec, h_spec, v_spec, v_spec,
                p_spec, p_spec, p_spec, p_spec,
                pl.BlockSpec(fc1_W.shape, lambda p, i: (0, 0)),
                pl.BlockSpec(fc1_b.shape, lambda p, i: (0,)),
                pl.BlockSpec(fc2_W.shape, lambda p, i: (0, 0)),
                pl.BlockSpec(fc2_b.shape, lambda p, i: (0,))],
      out_specs=pl.BlockSpec((G, C), lambda p, i: (0, 0)),
      out_shape=jax.ShapeDtypeStruct((G, C), jnp.float32),
      scratch_shapes=stats_t + [pltpu.VMEM((G, H), jnp.float32)],
  )(s2a, s2b, ht2a, ht2b, dis3, batch3, b2, gn2_w, gn2_b, gn2_a,
    fc1_W, fc1_b, fc2_W, fc2_b)
  return out
